# baseline (speedup 1.0000x reference)
import jax
import jax.numpy as jnp
from jax import lax
from jax.experimental import pallas as pl
from jax.experimental.pallas import tpu as pltpu
from jax.experimental.pallas import tpu_sc as plsc

VOCAB = 1000000
EMB = 64
B = 4096
F = 26
L = 20

NUM_WORKERS = 32
SEGS = B * F
SEGS_PER_W = SEGS // NUM_WORKERS
CHUNK = 32
ROWS = CHUNK * L
GATHER_SPLIT = 128
N_GATHER = ROWS // GATHER_SPLIT
N_CHUNKS = SEGS_PER_W // CHUNK
HALF = EMB // 2
HV = HALF // 16
IDX_BLK = 8


def _sc_body(x_hbm, tl_hbm, tr_hbm, out_hbm,
             idx_v, rows0l, rows0r, rows1l, rows1r, outa, outb,
             sem0, sem1, semoa, semob):
    nc = 2
    wid = lax.axis_index("s") * nc + lax.axis_index("c")

    def sync_idx_block(blk):
        idx_off = (wid * SEGS_PER_W + blk * IDX_BLK * CHUNK) * L
        slot = (blk % 2) * (IDX_BLK * ROWS)
        pltpu.sync_copy(x_hbm.at[pl.ds(idx_off, IDX_BLK * ROWS)],
                        idx_v.at[pl.ds(slot, IDX_BLK * ROWS)])

    def fire(itc, rows_l, rows_r, sem):
        base = (itc % (2 * IDX_BLK)) * ROWS
        for j in range(N_GATHER):
            isl = pl.ds(base + j * GATHER_SPLIT, GATHER_SPLIT)
            rsl = pl.ds(j * GATHER_SPLIT, GATHER_SPLIT)
            pltpu.async_copy(tl_hbm.at[idx_v.at[isl]], rows_l.at[rsl], sem)
            pltpu.async_copy(tr_hbm.at[idx_v.at[isl]], rows_r.at[rsl], sem)

    def drain(itc, rows_l, rows_r, sem):
        base = (itc % (2 * IDX_BLK)) * ROWS
        for j in range(N_GATHER):
            isl = pl.ds(base + j * GATHER_SPLIT, GATHER_SPLIT)
            rsl = pl.ds(j * GATHER_SPLIT, GATHER_SPLIT)
            pltpu.make_async_copy(
                tl_hbm.at[idx_v.at[isl]], rows_l.at[rsl], sem).wait()
            pltpu.make_async_copy(
                tr_hbm.at[idx_v.at[isl]], rows_r.at[rsl], sem).wait()

    def reduce(rows_l, rows_r, out_v):
        @plsc.parallel_loop(0, CHUNK, unroll=4)
        def seg_body(s):
            row0 = s * L
            for h, rows_v in ((0, rows_l), (1, rows_r)):
                for e in range(HV):
                    sl = pl.ds(e * 16, 16)
                    a = rows_v[row0, sl]
                    b = rows_v[row0 + 1, sl]
                    for l in range(2, L, 2):
                        a = a + rows_v[row0 + l, sl]
                        b = b + rows_v[row0 + l + 1, sl]
                    out_v[s, pl.ds(h * HALF + e * 16, 16)] = a + b

    def out_start(itc, out_v, semo):
        seg_base = wid * SEGS_PER_W + itc * CHUNK
        pltpu.async_copy(out_v, out_hbm.at[pl.ds(seg_base, CHUNK)], semo)

    def out_wait(out_v, semo):
        pltpu.make_async_copy(out_v, out_hbm.at[pl.ds(0, CHUNK)], semo).wait()

    sync_idx_block(0)
    fire(0, rows0l, rows0r, sem0)

    def pair_body(p, carry):
        it0 = 2 * p
        it1 = 2 * p + 1
        it2 = 2 * p + 2

        fire(it1, rows1l, rows1r, sem1)

        @pl.when(p > 0)
        def _():
            out_wait(outa, semoa)

        drain(it0, rows0l, rows0r, sem0)
        reduce(rows0l, rows0r, outa)
        out_start(it0, outa, semoa)

        @pl.when(jnp.logical_and(it2 % IDX_BLK == 0, it2 < N_CHUNKS))
        def _():
            sync_idx_block(it2 // IDX_BLK)

        @pl.when(it2 < N_CHUNKS)
        def _():
            fire(it2, rows0l, rows0r, sem0)

        @pl.when(p > 0)
        def _():
            out_wait(outb, semob)

        drain(it1, rows1l, rows1r, sem1)
        reduce(rows1l, rows1r, outb)
        out_start(it1, outb, semob)
        return carry

    lax.fori_loop(0, N_CHUNKS // 2, pair_body, 0)
    out_wait(outa, semoa)
    out_wait(outb, semob)


def kernel(x, table):
    x_flat = x.reshape(-1).astype(jnp.int32)
    tl = table[:, :HALF]
    tr = table[:, HALF:]
    mesh = plsc.VectorSubcoreMesh(core_axis_name="c", subcore_axis_name="s")
    out = pl.kernel(
        _sc_body,
        out_type=jax.ShapeDtypeStruct((SEGS, EMB), jnp.float32),
        mesh=mesh,
        scratch_types=[
            pltpu.VMEM((2 * IDX_BLK * ROWS,), jnp.int32),
            pltpu.VMEM((ROWS, HALF), jnp.float32),
            pltpu.VMEM((ROWS, HALF), jnp.float32),
            pltpu.VMEM((ROWS, HALF), jnp.float32),
            pltpu.VMEM((ROWS, HALF), jnp.float32),
            pltpu.VMEM((CHUNK, EMB), jnp.float32),
            pltpu.VMEM((CHUNK, EMB), jnp.float32),
            pltpu.SemaphoreType.DMA,
            pltpu.SemaphoreType.DMA,
            pltpu.SemaphoreType.DMA,
            pltpu.SemaphoreType.DMA,
        ],
        compiler_params=pltpu.CompilerParams(use_tc_tiling_on_sc=False),
    )(x_flat, tl, tr)
    return out.reshape(B, F * EMB)


# speedup vs baseline: 1.0694x; 1.0694x over previous
import jax
import jax.numpy as jnp
import numpy as np
from jax import lax
from jax.experimental import pallas as pl
from jax.experimental.pallas import tpu as pltpu
from jax.experimental.pallas import tpu_sc as plsc

VOCAB = 1000000
EMB = 64
B = 4096
F = 26
L = 20

NUM_WORKERS = 32
SEGS = B * F
SEGS_PER_W = SEGS // NUM_WORKERS
CHUNK = 32
ROWS = CHUNK * L
GATHER_SPLIT = 128
N_GATHER = ROWS // GATHER_SPLIT
N_CHUNKS = SEGS_PER_W // CHUNK
NG = EMB // 32
IDX_BLK = 8

_PERM = np.empty(EMB, dtype=np.int32)
for _g in range(NG):
    for _k in range(16):
        _PERM[32 * _g + 2 * _k] = 32 * _g + _k
        _PERM[32 * _g + 2 * _k + 1] = 32 * _g + 16 + _k


def _sc_body(x_hbm, table_hbm, out_hbm,
             idx_v, rows0, rows1, outa, outb, sem0, sem1, semoa, semob):
    nc = 2
    wid = lax.axis_index("s") * nc + lax.axis_index("c")

    def sync_idx_block(blk):
        idx_off = (wid * SEGS_PER_W + blk * IDX_BLK * CHUNK) * L
        slot = (blk % 2) * (IDX_BLK * ROWS)
        pltpu.sync_copy(x_hbm.at[pl.ds(idx_off, IDX_BLK * ROWS)],
                        idx_v.at[pl.ds(slot, IDX_BLK * ROWS)])

    def fire(itc, rows_v, sem):
        base = (itc % (2 * IDX_BLK)) * ROWS
        for j in range(N_GATHER):
            isl = pl.ds(base + j * GATHER_SPLIT, GATHER_SPLIT)
            rsl = pl.ds(j * GATHER_SPLIT, GATHER_SPLIT)
            pltpu.async_copy(table_hbm.at[idx_v.at[isl]], rows_v.at[rsl], sem)

    def drain(itc, rows_v, sem):
        base = (itc % (2 * IDX_BLK)) * ROWS
        for j in range(N_GATHER):
            isl = pl.ds(base + j * GATHER_SPLIT, GATHER_SPLIT)
            rsl = pl.ds(j * GATHER_SPLIT, GATHER_SPLIT)
            pltpu.make_async_copy(
                table_hbm.at[idx_v.at[isl]], rows_v.at[rsl], sem).wait()

    def reduce(rows_v, out_v):
        @plsc.parallel_loop(0, CHUNK, unroll=4)
        def seg_body(s):
            row0 = s * L
            for g in range(NG):
                sl = pl.ds(g * 32, 32)
                a0, b0 = plsc.unpack(rows_v[row0, sl],
                                     format=plsc.PackFormat.INTERLEAVED)
                a1, b1 = plsc.unpack(rows_v[row0 + 1, sl],
                                     format=plsc.PackFormat.INTERLEAVED)
                for l in range(2, L, 2):
                    xa, xb = plsc.unpack(rows_v[row0 + l, sl],
                                         format=plsc.PackFormat.INTERLEAVED)
                    a0 = a0 + xa
                    b0 = b0 + xb
                    ya, yb = plsc.unpack(rows_v[row0 + l + 1, sl],
                                         format=plsc.PackFormat.INTERLEAVED)
                    a1 = a1 + ya
                    b1 = b1 + yb
                out_v[s, pl.ds(g * 32, 16)] = a0 + a1
                out_v[s, pl.ds(g * 32 + 16, 16)] = b0 + b1

    def out_start(itc, out_v, semo):
        seg_base = wid * SEGS_PER_W + itc * CHUNK
        pltpu.async_copy(out_v, out_hbm.at[pl.ds(seg_base, CHUNK)], semo)

    def out_wait(out_v, semo):
        pltpu.make_async_copy(out_v, out_hbm.at[pl.ds(0, CHUNK)], semo).wait()

    sync_idx_block(0)
    fire(0, rows0, sem0)

    def pair_body(p, carry):
        it0 = 2 * p
        it1 = 2 * p + 1
        it2 = 2 * p + 2

        fire(it1, rows1, sem1)

        @pl.when(p > 0)
        def _():
            out_wait(outa, semoa)

        drain(it0, rows0, sem0)
        reduce(rows0, outa)
        out_start(it0, outa, semoa)

        @pl.when(jnp.logical_and(it2 % IDX_BLK == 0, it2 < N_CHUNKS))
        def _():
            sync_idx_block(it2 // IDX_BLK)

        @pl.when(it2 < N_CHUNKS)
        def _():
            fire(it2, rows0, sem0)

        @pl.when(p > 0)
        def _():
            out_wait(outb, semob)

        drain(it1, rows1, sem1)
        reduce(rows1, outb)
        out_start(it1, outb, semob)
        return carry

    lax.fori_loop(0, N_CHUNKS // 2, pair_body, 0)
    out_wait(outa, semoa)
    out_wait(outb, semob)


def kernel(x, table):
    x_flat = x.reshape(-1).astype(jnp.int32)
    t16 = table[:, jnp.asarray(_PERM)].astype(jnp.bfloat16)
    mesh = plsc.VectorSubcoreMesh(core_axis_name="c", subcore_axis_name="s")
    out = pl.kernel(
        _sc_body,
        out_type=jax.ShapeDtypeStruct((SEGS, EMB), jnp.float32),
        mesh=mesh,
        scratch_types=[
            pltpu.VMEM((2 * IDX_BLK * ROWS,), jnp.int32),
            pltpu.VMEM((ROWS, EMB), jnp.bfloat16),
            pltpu.VMEM((ROWS, EMB), jnp.bfloat16),
            pltpu.VMEM((CHUNK, EMB), jnp.float32),
            pltpu.VMEM((CHUNK, EMB), jnp.float32),
            pltpu.SemaphoreType.DMA,
            pltpu.SemaphoreType.DMA,
            pltpu.SemaphoreType.DMA,
            pltpu.SemaphoreType.DMA,
        ],
        compiler_params=pltpu.CompilerParams(use_tc_tiling_on_sc=False,
                                             needs_layout_passes=False),
    )(x_flat, t16)
    return out.reshape(B, F * EMB)


# speedup vs baseline: 1.9010x; 1.7776x over previous
import jax
import jax.numpy as jnp
from jax import lax
from jax.experimental import pallas as pl
from jax.experimental.pallas import tpu as pltpu
from jax.experimental.pallas import tpu_sc as plsc

VOCAB = 1000000
EMB = 64
B = 4096
F = 26
L = 20

NUM_WORKERS = 32
SEGS = B * F
SEGS_PER_W = SEGS // NUM_WORKERS
CHUNK = 32
ROWS = CHUNK * L
GATHER_SPLIT = 128
N_GATHER = ROWS // GATHER_SPLIT
N_CHUNKS = SEGS_PER_W // CHUNK
EV = EMB // 16
IDX_BLK = 8


def _sc_body(x_hbm, table_hbm, out_hbm,
             idx_v, rows0, rows1, outa, outb, sem0, sem1, semoa, semob):
    nc = 2
    wid = lax.axis_index("s") * nc + lax.axis_index("c")

    def sync_idx_block(blk):
        idx_off = (wid * SEGS_PER_W + blk * IDX_BLK * CHUNK) * L
        slot = (blk % 2) * (IDX_BLK * ROWS)
        pltpu.sync_copy(x_hbm.at[pl.ds(idx_off, IDX_BLK * ROWS)],
                        idx_v.at[pl.ds(slot, IDX_BLK * ROWS)])

    def fire(itc, rows_v, sem):
        base = (itc % (2 * IDX_BLK)) * ROWS
        for j in range(N_GATHER):
            isl = pl.ds(base + j * GATHER_SPLIT, GATHER_SPLIT)
            rsl = pl.ds(j * GATHER_SPLIT, GATHER_SPLIT)
            pltpu.async_copy(table_hbm.at[idx_v.at[isl]], rows_v.at[rsl], sem)

    def drain(itc, rows_v, sem):
        base = (itc % (2 * IDX_BLK)) * ROWS
        for j in range(N_GATHER):
            isl = pl.ds(base + j * GATHER_SPLIT, GATHER_SPLIT)
            rsl = pl.ds(j * GATHER_SPLIT, GATHER_SPLIT)
            pltpu.make_async_copy(
                table_hbm.at[idx_v.at[isl]], rows_v.at[rsl], sem).wait()

    def reduce(rows_v, out_v):
        @plsc.parallel_loop(0, CHUNK, unroll=4)
        def seg_body(s):
            row0 = s * L
            for e in range(EV):
                sl = pl.ds(e * 16, 16)
                a = rows_v[row0, sl]
                b = rows_v[row0 + 1, sl]
                for l in range(2, L, 2):
                    a = a + rows_v[row0 + l, sl]
                    b = b + rows_v[row0 + l + 1, sl]
                out_v[s, sl] = a + b

    def out_start(itc, out_v, semo):
        seg_base = wid * SEGS_PER_W + itc * CHUNK
        pltpu.async_copy(out_v, out_hbm.at[pl.ds(seg_base, CHUNK)], semo)

    def out_wait(out_v, semo):
        pltpu.make_async_copy(out_v, out_hbm.at[pl.ds(0, CHUNK)], semo).wait()

    sync_idx_block(0)
    fire(0, rows0, sem0)

    def pair_body(p, carry):
        it0 = 2 * p
        it1 = 2 * p + 1
        it2 = 2 * p + 2

        fire(it1, rows1, sem1)

        @pl.when(p > 0)
        def _():
            out_wait(outa, semoa)

        drain(it0, rows0, sem0)
        reduce(rows0, outa)
        out_start(it0, outa, semoa)

        @pl.when(jnp.logical_and(it2 % IDX_BLK == 0, it2 < N_CHUNKS))
        def _():
            sync_idx_block(it2 // IDX_BLK)

        @pl.when(it2 < N_CHUNKS)
        def _():
            fire(it2, rows0, sem0)

        @pl.when(p > 0)
        def _():
            out_wait(outb, semob)

        drain(it1, rows1, sem1)
        reduce(rows1, outb)
        out_start(it1, outb, semob)
        return carry

    lax.fori_loop(0, N_CHUNKS // 2, pair_body, 0)
    out_wait(outa, semoa)
    out_wait(outb, semob)


def kernel(x, table):
    x_flat = x.reshape(-1).astype(jnp.int32)
    mesh = plsc.VectorSubcoreMesh(core_axis_name="c", subcore_axis_name="s")
    out = pl.kernel(
        _sc_body,
        out_type=jax.ShapeDtypeStruct((SEGS, EMB), jnp.float32),
        mesh=mesh,
        scratch_types=[
            pltpu.VMEM((2 * IDX_BLK * ROWS,), jnp.int32),
            pltpu.VMEM((ROWS, EMB), jnp.float32),
            pltpu.VMEM((ROWS, EMB), jnp.float32),
            pltpu.VMEM((CHUNK, EMB), jnp.float32),
            pltpu.VMEM((CHUNK, EMB), jnp.float32),
            pltpu.SemaphoreType.DMA,
            pltpu.SemaphoreType.DMA,
            pltpu.SemaphoreType.DMA,
            pltpu.SemaphoreType.DMA,
        ],
        compiler_params=pltpu.CompilerParams(use_tc_tiling_on_sc=False),
    )(x_flat, table)
    return out.reshape(B, F * EMB)
